# baseline (speedup 1.0000x reference)
import jax
import jax.numpy as jnp
from jax.experimental import pallas as pl
from jax.experimental.pallas import tpu as pltpu


def _dma3(len_ref, r_ref, t_ref, p_ref, ro_ref, to_ref, po_ref,
          sem_r, sem_t, sem_p):
    idx = len_ref[0] - 1
    cp_r = pltpu.make_async_copy(r_ref.at[idx], ro_ref, sem_r)
    cp_t = pltpu.make_async_copy(t_ref.at[idx], to_ref, sem_t)
    cp_p = pltpu.make_async_copy(p_ref.at[idx], po_ref, sem_p)
    cp_t.start()
    cp_r.start()
    cp_p.start()
    cp_t.wait()
    cp_r.wait()
    cp_p.wait()


def kernel(rule_prob, token_prob, reference_prob, length):
    L, B, R = rule_prob.shape
    V = token_prob.shape[2]
    M = reference_prob.shape[2]
    tok_t = token_prob.transpose(0, 2, 1)
    ref_t = reference_prob.transpose(0, 2, 1)

    r, t_t, p_t = pl.pallas_call(
        _dma3,
        in_specs=[
            pl.BlockSpec(memory_space=pltpu.SMEM),
            pl.BlockSpec(memory_space=pl.ANY),
            pl.BlockSpec(memory_space=pl.ANY),
            pl.BlockSpec(memory_space=pl.ANY),
        ],
        out_specs=[
            pl.BlockSpec(memory_space=pl.ANY),
            pl.BlockSpec(memory_space=pl.ANY),
            pl.BlockSpec(memory_space=pl.ANY),
        ],
        out_shape=[
            jax.ShapeDtypeStruct((B, R), jnp.float32),
            jax.ShapeDtypeStruct((V, B), jnp.float32),
            jax.ShapeDtypeStruct((M, B), jnp.float32),
        ],
        scratch_shapes=[pltpu.SemaphoreType.DMA] * 3,
    )(length, rule_prob, tok_t, ref_t)
    return (r, t_t.T, p_t.T)


# speedup vs baseline: 20.3381x; 20.3381x over previous
import jax
import jax.numpy as jnp
from jax.experimental import pallas as pl
from jax.experimental.pallas import tpu as pltpu

_G = 5


def _copy3(idx_ref, r_ref, t_ref, p_ref, ro_ref, to_ref, po_ref):
    del idx_ref
    to_ref[...] = t_ref[0]

    @pl.when(pl.program_id(0) == 0)
    def _():
        ro_ref[...] = r_ref[0]
        po_ref[...] = p_ref[0]


def kernel(rule_prob, token_prob, reference_prob, length):
    L, B, R = rule_prob.shape
    V = token_prob.shape[2]
    M = reference_prob.shape[2]
    idx = (length - 1).astype(jnp.int32)
    tok_t = token_prob.transpose(0, 2, 1)
    ref_t = reference_prob.transpose(0, 2, 1)
    Vb = V // _G

    grid_spec = pltpu.PrefetchScalarGridSpec(
        num_scalar_prefetch=1,
        grid=(_G,),
        in_specs=[
            pl.BlockSpec((1, B, R), lambda g, idx_ref: (idx_ref[0], 0, 0)),
            pl.BlockSpec((1, Vb, B), lambda g, idx_ref: (idx_ref[0], g, 0)),
            pl.BlockSpec((1, M, B), lambda g, idx_ref: (idx_ref[0], 0, 0)),
        ],
        out_specs=[
            pl.BlockSpec((B, R), lambda g, idx_ref: (0, 0)),
            pl.BlockSpec((Vb, B), lambda g, idx_ref: (g, 0)),
            pl.BlockSpec((M, B), lambda g, idx_ref: (0, 0)),
        ],
    )
    r, t_t, p_t = pl.pallas_call(
        _copy3,
        grid_spec=grid_spec,
        out_shape=[
            jax.ShapeDtypeStruct((B, R), jnp.float32),
            jax.ShapeDtypeStruct((V, B), jnp.float32),
            jax.ShapeDtypeStruct((M, B), jnp.float32),
        ],
    )(idx, rule_prob, tok_t, ref_t)
    return (r, t_t.T, p_t.T)
